# baseline (speedup 1.0000x reference)
import functools

import jax
import jax.numpy as jnp
import numpy as np
from jax import lax
from jax.experimental import pallas as pl
from jax.experimental.pallas import tpu as pltpu
from jax.experimental.pallas import tpu_sc as plsc

_F = 26
_V = 100000
_D = 16
_B = 4096
_NPAIR = (_F * (_F - 1)) // 2
_NW = 32
_IDXW = 128
_G = 8
_CW = _G * _IDXW
_CPP = _B // _CW
_NCHUNK = _NPAIR * _CPP
_BASE = _NCHUNK // _NW
_EXTRA = _NCHUNK % _NW


_NR = _F * _V
_QUOTA = 81256
_LAST = _NR - 31 * _QUOTA


def _sizes(total):
    out, c = [], 2048
    while total:
        while c > total:
            c //= 2
        out.append(c)
        total -= c
    return out


def _sc_detile(table2d):
    mesh = plsc.VectorSubcoreMesh(core_axis_name="c", subcore_axis_name="s")

    @functools.partial(
        pl.kernel,
        mesh=mesh,
        out_type=jax.ShapeDtypeStruct((_NR, _D), jnp.float32),
        scratch_types=[pltpu.SemaphoreType.DMA],
        compiler_params=pltpu.CompilerParams(
            use_tc_tiling_on_sc=True, needs_layout_passes=False),
    )
    def k(src_hbm, dst_hbm, sem):
        wid = lax.axis_index("s") * 2 + lax.axis_index("c")
        base = wid * _QUOTA

        def run(sizes):
            cps, off = [], 0
            for s in sizes:
                cps.append(pltpu.make_async_copy(
                    src_hbm.at[pl.ds(base + off, s), :],
                    dst_hbm.at[pl.ds(base + off, s), :], sem))
                off += s
            for cp in cps:
                cp.start()
            for cp in cps:
                cp.wait()

        @pl.when(wid < 31)
        def _():
            run(_sizes(_QUOTA))

        @pl.when(wid == 31)
        def _():
            run(_sizes(_LAST))

    return k(table2d)


def _sc_ffm(idxa, idxb, table):
    mesh = plsc.VectorSubcoreMesh(core_axis_name="c", subcore_axis_name="s")

    @functools.partial(
        pl.kernel,
        mesh=mesh,
        out_type=jax.ShapeDtypeStruct((_NPAIR, _D, _B), jnp.float32),
        scratch_types=[
            pltpu.VMEM((2, _CW), jnp.int32),
            pltpu.VMEM((2, _CW), jnp.int32),
            pltpu.VMEM((2 * _CW, _D), jnp.float32),
            pltpu.VMEM((2 * _CW, _D), jnp.float32),
            pltpu.VMEM((2 * _D, _CW + 1), jnp.float32),
            pltpu.SemaphoreType.DMA,
            pltpu.SemaphoreType.DMA,
            pltpu.SemaphoreType.DMA,
            pltpu.SemaphoreType.DMA,
            pltpu.SemaphoreType.DMA,
        ],
        compiler_params=pltpu.CompilerParams(
            use_tc_tiling_on_sc=False, needs_layout_passes=False),
    )
    def k(idxa_hbm, idxb_hbm, table_hbm, out_hbm,
          idxa_v, idxb_v, ra_v, rb_v, out_v, semi, semg0, semg1, semo0, semo1):
        wid = lax.axis_index("s") * 2 + lax.axis_index("c")
        nchunk = _BASE + jnp.where(wid < _EXTRA, 1, 0)
        didx = lax.iota(jnp.int32, _D)
        semg = (semg0, semg1)
        semo = (semo0, semo1)

        def cc_of(ci):
            return wid + ci * _NW

        def idx_copy(ci, slot, fire):
            off = cc_of(ci) * _CW
            for src, dst in ((idxa_hbm, idxa_v), (idxb_hbm, idxb_v)):
                cp = pltpu.make_async_copy(
                    src.at[pl.ds(off, _CW)], dst.at[slot], semi)
                if fire:
                    cp.start()
                else:
                    cp.wait()

        def gathers(slot, fire):
            for g in range(_G):
                s = pl.ds(g * _IDXW, _IDXW)
                d = pl.ds(slot * _CW + g * _IDXW, _IDXW)
                for iv, rv in ((idxa_v, ra_v), (idxb_v, rb_v)):
                    cp = pltpu.make_async_copy(
                        table_hbm.at[iv.at[slot].at[s]], rv.at[d], semg[slot])
                    if fire:
                        cp.start()
                    else:
                        cp.wait()

        def out_fire(ci, slot):
            cc = cc_of(ci)
            p = cc // _CPP
            sub = cc % _CPP
            pltpu.make_async_copy(
                out_v.at[pl.ds(slot * _D, _D), pl.ds(0, _CW)],
                out_hbm.at[p, :, pl.ds(sub * _CW, _CW)],
                semo[slot]).start()

        def out_drain(slot):
            pltpu.make_async_copy(
                out_v.at[pl.ds(slot * _D, _D), pl.ds(0, _CW)],
                out_hbm.at[0, :, pl.ds(0, _CW)],
                semo[slot]).wait()

        def compute(slot):
            rowv = didx + slot * _D
            rbase = slot * _CW

            def prod8(q, c):
                l = q * 8
                for k_ in range(8):
                    mv = didx * 0 + (l + k_)
                    a = plsc.load_gather(ra_v, [mv + rbase, didx])
                    b = plsc.load_gather(rb_v, [mv + rbase, didx])
                    plsc.store_scatter(out_v, [rowv, mv], a * b)
                return c

            lax.fori_loop(0, _CW // 8, prod8, 0)

        idx_copy(0, 0, True)
        idx_copy(1, 1, True)
        idx_copy(0, 0, False)
        gathers(0, True)

        def body(ci, carry):
            for s_ in (0, 1):
                @pl.when(ci % 2 == s_)
                def _(s_=s_):
                    cur, oth = s_, 1 - s_

                    @pl.when(ci + 1 < nchunk)
                    def _():
                        idx_copy(ci + 1, oth, False)
                        gathers(oth, True)

                    gathers(cur, False)
                    @pl.when(ci + 2 < nchunk)
                    def _():
                        idx_copy(ci + 2, cur, True)

                    @pl.when(ci >= 2)
                    def _():
                        out_drain(cur)

                    compute(cur)
                    out_fire(ci, cur)
            return carry

        lax.fori_loop(0, nchunk, body, 0)

        out_drain(0)
        out_drain(1)

    return k(idxa, idxb, table)


def kernel(x, tables, offsets):
    xi_t = (x + offsets[None, :]).T
    iu, ju = np.triu_indices(_F, k=1)
    iu = jnp.asarray(iu, jnp.int32)
    ju = jnp.asarray(ju, jnp.int32)
    idxa = (xi_t[iu] + (ju * _V)[:, None]).reshape(_NPAIR * _B)
    idxb = (xi_t[ju] + (iu * _V)[:, None]).reshape(_NPAIR * _B)
    table = _sc_detile(tables.reshape(_F * _V, _D))
    out = _sc_ffm(idxa, idxb, table)
    return jnp.transpose(out, (2, 0, 1))


# speedup vs baseline: 29.9816x; 29.9816x over previous
import functools

import jax
import jax.numpy as jnp
import numpy as np
from jax import lax
from jax.experimental import pallas as pl
from jax.experimental.pallas import tpu as pltpu
from jax.experimental.pallas import tpu_sc as plsc

_F = 26
_V = 100000
_D = 16
_B = 4096
_NPAIR = (_F * (_F - 1)) // 2
_NW = 32
_IDXW = 128
_G = 8
_CW = _G * _IDXW
_CPP = _B // _CW
_NCHUNK = _NPAIR * _CPP
_BASE = _NCHUNK // _NW
_EXTRA = _NCHUNK % _NW


def _sc_ffm(idxa, idxb, table):
    mesh = plsc.VectorSubcoreMesh(core_axis_name="c", subcore_axis_name="s")

    @functools.partial(
        pl.kernel,
        mesh=mesh,
        out_type=jax.ShapeDtypeStruct((_NPAIR, _D, _B), jnp.float32),
        scratch_types=[
            pltpu.VMEM((2, _CW), jnp.int32),
            pltpu.VMEM((2, _CW), jnp.int32),
            pltpu.VMEM((2 * _CW, _D), jnp.float32),
            pltpu.VMEM((2 * _CW, _D), jnp.float32),
            pltpu.VMEM((2 * _D, _CW + 1), jnp.float32),
            pltpu.SemaphoreType.DMA,
            pltpu.SemaphoreType.DMA,
            pltpu.SemaphoreType.DMA,
            pltpu.SemaphoreType.DMA,
            pltpu.SemaphoreType.DMA,
        ],
        compiler_params=pltpu.CompilerParams(
            use_tc_tiling_on_sc=False, needs_layout_passes=False),
    )
    def k(idxa_hbm, idxb_hbm, table_hbm, out_hbm,
          idxa_v, idxb_v, ra_v, rb_v, out_v, semi, semg0, semg1, semo0, semo1):
        wid = lax.axis_index("s") * 2 + lax.axis_index("c")
        nchunk = _BASE + jnp.where(wid < _EXTRA, 1, 0)
        didx = lax.iota(jnp.int32, _D)
        semg = (semg0, semg1)
        semo = (semo0, semo1)

        def cc_of(ci):
            return wid + ci * _NW

        def idx_copy(ci, slot, fire):
            off = cc_of(ci) * _CW
            for src, dst in ((idxa_hbm, idxa_v), (idxb_hbm, idxb_v)):
                cp = pltpu.make_async_copy(
                    src.at[pl.ds(off, _CW)], dst.at[slot], semi)
                if fire:
                    cp.start()
                else:
                    cp.wait()

        def gathers(slot, fire):
            for g in range(_G):
                s = pl.ds(g * _IDXW, _IDXW)
                d = pl.ds(slot * _CW + g * _IDXW, _IDXW)
                for iv, rv in ((idxa_v, ra_v), (idxb_v, rb_v)):
                    cp = pltpu.make_async_copy(
                        table_hbm.at[iv.at[slot].at[s]], rv.at[d], semg[slot])
                    if fire:
                        cp.start()
                    else:
                        cp.wait()

        def out_fire(ci, slot):
            cc = cc_of(ci)
            p = cc // _CPP
            sub = cc % _CPP
            pltpu.make_async_copy(
                out_v.at[pl.ds(slot * _D, _D), pl.ds(0, _CW)],
                out_hbm.at[p, :, pl.ds(sub * _CW, _CW)],
                semo[slot]).start()

        def out_drain(slot):
            pltpu.make_async_copy(
                out_v.at[pl.ds(slot * _D, _D), pl.ds(0, _CW)],
                out_hbm.at[0, :, pl.ds(0, _CW)],
                semo[slot]).wait()

        def compute(slot):
            rowv = didx + slot * _D
            rbase = slot * _CW

            def prod8(q, c):
                l = q * 8
                for k_ in range(8):
                    mv = didx * 0 + (l + k_)
                    a = plsc.load_gather(ra_v, [mv + rbase, didx])
                    b = plsc.load_gather(rb_v, [mv + rbase, didx])
                    plsc.store_scatter(out_v, [rowv, mv], a * b)
                return c

            lax.fori_loop(0, _CW // 8, prod8, 0)

        idx_copy(0, 0, True)
        idx_copy(1, 1, True)
        idx_copy(0, 0, False)
        gathers(0, True)

        def body(ci, carry):
            for s_ in (0, 1):
                @pl.when(ci % 2 == s_)
                def _(s_=s_):
                    cur, oth = s_, 1 - s_

                    @pl.when(ci + 1 < nchunk)
                    def _():
                        idx_copy(ci + 1, oth, False)
                        gathers(oth, True)

                    gathers(cur, False)
                    @pl.when(ci + 2 < nchunk)
                    def _():
                        idx_copy(ci + 2, cur, True)

                    @pl.when(ci >= 2)
                    def _():
                        out_drain(cur)

                    compute(cur)
                    out_fire(ci, cur)
            return carry

        lax.fori_loop(0, nchunk, body, 0)

        out_drain(0)
        out_drain(1)

    return k(idxa, idxb, table)


def kernel(x, tables, offsets):
    xi_t = (x + offsets[None, :]).T
    iu, ju = np.triu_indices(_F, k=1)
    iu = jnp.asarray(iu, jnp.int32)
    ju = jnp.asarray(ju, jnp.int32)
    idxa = (xi_t[iu] + (ju * _V)[:, None]).reshape(_NPAIR * _B)
    idxb = (xi_t[ju] + (iu * _V)[:, None]).reshape(_NPAIR * _B)
    table = tables.reshape(_F * _V, _D)
    out = _sc_ffm(idxa, idxb, table)
    return jnp.transpose(out, (2, 0, 1))
